# Initial kernel scaffold; baseline (speedup 1.0000x reference)
import jax
import jax.numpy as jnp
from jax.experimental import pallas as pl


def kernel(neighbour_index, neighbour_distances, node_features, W_lin, b_lin, W_f1, b_f1, W_f2, b_f2, W_m1, b_m1, W_m2, b_m2):
    raise NotImplementedError("write your pallas kernel here")


# speedup vs baseline: 2.4290x; 2.4290x over previous
import functools

import jax
import jax.numpy as jnp
from jax import lax
from jax.experimental import pallas as pl
from jax.experimental.pallas import tpu as pltpu
from jax.experimental.pallas import tpu_sc as plsc

CUTOFF = 10.0

NC = 2
NS = 16
LANES = 16


def _ssp(x):
    return jnp.maximum(x, 0.0) + jnp.log1p(jnp.exp(-jnp.abs(x))) - 0.6931471805599453


def _h_body(x_ref, w_ref, b_ref, o_ref):
    o_ref[...] = (
        jnp.dot(x_ref[...], w_ref[...], preferred_element_type=jnp.float32)
        + b_ref[...]
    )


def _compute_h(x, w_t, b, block_n):
    n, f = x.shape
    return pl.pallas_call(
        _h_body,
        grid=(n // block_n,),
        in_specs=[
            pl.BlockSpec((block_n, f), lambda i: (i, 0)),
            pl.BlockSpec((f, f), lambda i: (0, 0)),
            pl.BlockSpec((1, f), lambda i: (0, 0)),
        ],
        out_specs=pl.BlockSpec((block_n, f), lambda i: (i, 0)),
        out_shape=jax.ShapeDtypeStruct((n, f), jnp.float32),
    )(x, w_t, b)


def _filters_body(g_count, d_ref, w1_ref, b1_ref, w2_ref, b2_ref, o_ref):
    step = CUTOFF / (g_count - 1)
    coeff = -0.5 / step**2
    offset = lax.broadcasted_iota(jnp.int32, (1, g_count), 1).astype(jnp.float32) * step
    expanded = jnp.exp(coeff * (d_ref[...] - offset) ** 2)
    t = _ssp(
        jnp.dot(expanded, w1_ref[...], preferred_element_type=jnp.float32)
        + b1_ref[...]
    )
    o_ref[...] = (
        jnp.dot(t, w2_ref[...], preferred_element_type=jnp.float32) + b2_ref[...]
    )


def _compute_filters(d2, w1_t, b1, w2_t, b2, block_e):
    e = d2.shape[0]
    g = w1_t.shape[0]
    f = w1_t.shape[1]
    return pl.pallas_call(
        functools.partial(_filters_body, g),
        grid=(e // block_e,),
        in_specs=[
            pl.BlockSpec((block_e, 1), lambda i: (i, 0)),
            pl.BlockSpec((g, f), lambda i: (0, 0)),
            pl.BlockSpec((1, f), lambda i: (0, 0)),
            pl.BlockSpec((f, f), lambda i: (0, 0)),
            pl.BlockSpec((1, f), lambda i: (0, 0)),
        ],
        out_specs=pl.BlockSpec((block_e, f), lambda i: (i, 0)),
        out_shape=jax.ShapeDtypeStruct((e, f), jnp.float32),
    )(d2, w1_t, b1, w2_t, b2)


def _cfconv_sc(src2, dst2, h, filters, zeros_nf):
    np_, f = zeros_nf.shape
    n_chunks, ch = src2.shape
    rows_per_sub = np_ // NS
    chunks_per_core = n_chunks // NC
    mesh = plsc.VectorSubcoreMesh(core_axis_name="c", subcore_axis_name="s")

    @functools.partial(
        pl.kernel,
        out_type=jax.ShapeDtypeStruct((NC, np_, f), jnp.float32),
        mesh=mesh,
        scratch_types=[
            pltpu.VMEM((1, ch), jnp.int32),
            pltpu.VMEM((1, ch), jnp.int32),
            pltpu.VMEM((ch, f), jnp.float32),
            pltpu.VMEM((ch, f), jnp.float32),
            pltpu.VMEM_SHARED((np_, f), jnp.float32),
            pltpu.SemaphoreType.DMA,
        ],
    )
    def sc_kernel(src_hbm, dst_hbm, h_hbm, filt_hbm, zero_hbm, out_hbm,
                  src_v, dst_v, rows_v, filt_v, acc_shared, sem):
        cid = lax.axis_index("c")
        sid = lax.axis_index("s")

        row0 = sid * rows_per_sub
        pltpu.sync_copy(
            zero_hbm.at[pl.ds(row0, rows_per_sub)],
            acc_shared.at[pl.ds(row0, rows_per_sub)],
        )
        plsc.subcore_barrier()

        lo = cid * chunks_per_core + sid
        hi = (cid + 1) * chunks_per_core

        @pl.loop(lo, hi, step=NS)
        def _(c):
            pltpu.sync_copy(src_hbm.at[pl.ds(c, 1)], src_v)
            pltpu.sync_copy(dst_hbm.at[pl.ds(c, 1)], dst_v)
            gather = pltpu.async_copy(h_hbm.at[src_v.at[0]], rows_v, sem)
            pltpu.sync_copy(filt_hbm.at[pl.ds(c * ch, ch)], filt_v)
            gather.wait()

            @pl.loop(0, ch)
            def _(r):
                @pl.loop(0, f, step=LANES)
                def _(k):
                    rows_v[r, pl.ds(k, LANES)] = (
                        rows_v[r, pl.ds(k, LANES)] * filt_v[r, pl.ds(k, LANES)]
                    )

            pltpu.sync_copy(rows_v, acc_shared.at[dst_v.at[0]], add=True)

        plsc.subcore_barrier()
        pltpu.sync_copy(
            acc_shared.at[pl.ds(row0, rows_per_sub)],
            out_hbm.at[cid, pl.ds(row0, rows_per_sub)],
        )

    return sc_kernel(src2, dst2, h, filters, zeros_nf)


def _out_body(p_ref, w1_ref, b1_ref, w2_ref, b2_ref, o_ref):
    agg = p_ref[0] + p_ref[1]
    t = _ssp(
        jnp.dot(agg, w1_ref[...], preferred_element_type=jnp.float32)
        + b1_ref[...]
    )
    o_ref[...] = (
        jnp.dot(t, w2_ref[...], preferred_element_type=jnp.float32) + b2_ref[...]
    )


def _compute_out(partials, n, w1_t, b1, w2_t, b2, block_n):
    _, _, f = partials.shape
    return pl.pallas_call(
        _out_body,
        grid=(n // block_n,),
        in_specs=[
            pl.BlockSpec((NC, block_n, f), lambda i: (0, i, 0)),
            pl.BlockSpec((f, f), lambda i: (0, 0)),
            pl.BlockSpec((1, f), lambda i: (0, 0)),
            pl.BlockSpec((f, f), lambda i: (0, 0)),
            pl.BlockSpec((1, f), lambda i: (0, 0)),
        ],
        out_specs=pl.BlockSpec((block_n, f), lambda i: (i, 0)),
        out_shape=jax.ShapeDtypeStruct((n, f), jnp.float32),
    )(partials, w1_t, b1, w2_t, b2)


def kernel(neighbour_index, neighbour_distances, node_features,
           W_lin, b_lin, W_f1, b_f1, W_f2, b_f2, W_m1, b_m1, W_m2, b_m2):
    n, f = node_features.shape
    e = neighbour_distances.shape[0]
    ch = 128

    h = _compute_h(node_features, W_lin.T, b_lin.reshape(1, f), block_n=1000)
    filters = _compute_filters(
        neighbour_distances.reshape(e, 1),
        W_f1.T, b_f1.reshape(1, f), W_f2.T, b_f2.reshape(1, f),
        block_e=2000,
    )
    src2 = neighbour_index[0].reshape(e // ch, ch)
    dst2 = neighbour_index[1].reshape(e // ch, ch)
    n_pad = ((n + 8 * NS - 1) // (8 * NS)) * (8 * NS)
    zeros_nf = jnp.zeros((n_pad, f), jnp.float32)
    partials = _cfconv_sc(src2, dst2, h, filters, zeros_nf)
    return _compute_out(
        partials, n, W_m1.T, b_m1.reshape(1, f), W_m2.T, b_m2.reshape(1, f),
        block_n=1000,
    )
